# baseline (speedup 1.0000x reference)
import functools

import jax
import jax.numpy as jnp
from jax import lax
from jax.experimental import pallas as pl
from jax.experimental.pallas import tpu as pltpu
from jax.experimental.pallas import tpu_sc as plsc

FEATURE_NUM = 1040000
LATENT_DIM = 16
FIELD_NUM = 26
BATCH = 16384

NC = 2
NS = 16
NW = NC * NS
BW = BATCH // NW
GATHER_ROWS = 128
KJ = BW // GATHER_ROWS
NBLK = BW // 16
NT = BATCH // 128
TW = NT // NW

_mesh = plsc.VectorSubcoreMesh(core_axis_name="c", subcore_axis_name="s")

RT = FEATURE_NUM // 128


@functools.partial(
    pl.kernel,
    out_type=jax.ShapeDtypeStruct((FEATURE_NUM // 8, 128), jnp.float32),
    mesh=_mesh,
    compiler_params=pltpu.CompilerParams(
        needs_layout_passes=False, use_tc_tiling_on_sc=False
    ),
    scratch_types=[
        pltpu.VMEM((2, 8, 128), jnp.float32),
        pltpu.VMEM((16, 128), jnp.float32),
    ],
)
def _sc_relayout(emb4d_hbm, out_hbm, in_v, out_v):
    wid = lax.axis_index("s") * NC + lax.axis_index("c")
    t_lo = wid * RT // NW
    t_hi = (wid + 1) * RT // NW

    iota = lax.iota(jnp.int32, 16)
    gv = lax.shift_right_logical(iota, 3)
    ddv = lax.bitwise_and(iota, jnp.int32(7))

    def tile_body(t, carry):
        pltpu.sync_copy(emb4d_hbm.at[:, t], in_v)
        for j in range(16):
            for u in range(8):
                cv = jnp.full((16,), 8 * j + u, jnp.int32)
                out_v[j, pl.ds(u * 16, 16)] = plsc.load_gather(
                    in_v, [gv, ddv, cv]
                )
        pltpu.sync_copy(out_v, out_hbm.at[pl.ds(t * 16, 16)])
        return carry

    lax.fori_loop(t_lo, t_hi, tile_body, 0)


@functools.partial(
    pl.kernel,
    out_type=jax.ShapeDtypeStruct(
        (FIELD_NUM, LATENT_DIM // 8, NT, 8, 128), jnp.float32
    ),
    mesh=_mesh,
    compiler_params=pltpu.CompilerParams(
        needs_layout_passes=False, use_tc_tiling_on_sc=False
    ),
    scratch_types=[
        pltpu.VMEM((BW,), jnp.int32),
        pltpu.VMEM((BW, LATENT_DIM), jnp.float32),
        pltpu.VMEM((2, TW, 8, 128), jnp.float32),
        pltpu.VMEM((FIELD_NUM, 16), jnp.float32),
        pltpu.SemaphoreType.DMA,
    ],
)
def _sc_embed(xt_hbm, thr_hbm, table_hbm, out_hbm, idx_v, rows_v, trans_v,
              thr_v, sem):
    wid = lax.axis_index("s") * NC + lax.axis_index("c")
    b0 = wid * BW
    t0 = wid * TW

    pltpu.sync_copy(thr_hbm, thr_v)

    def field_body(f, carry):
        pltpu.sync_copy(xt_hbm.at[f, pl.ds(b0, BW)], idx_v)

        descs = []
        for j in range(KJ):
            descs.append(
                pltpu.async_copy(
                    table_hbm.at[idx_v.at[pl.ds(j * GATHER_ROWS, GATHER_ROWS)]],
                    rows_v.at[pl.ds(j * GATHER_ROWS, GATHER_ROWS)],
                    sem,
                )
            )
        for d in descs:
            d.wait()

        t_vec = thr_v[f, :]

        def blk_body(blk, rcarry):
            ridx = blk * 16 + lax.iota(jnp.int32, 16)
            sums = jnp.zeros((16,), jnp.float32)
            cols = []
            for d in range(LATENT_DIM):
                cidx = jnp.full((16,), d, jnp.int32)
                col = plsc.load_gather(rows_v, [ridx, cidx])
                cols.append(col)
                sums = sums + jnp.abs(col)
            m = ((sums - t_vec) > 0).astype(jnp.float32)
            tl = blk // 8
            rr0 = (blk % 8) * 16
            for d in range(LATENT_DIM):
                trans_v[d // 8, tl, d % 8, pl.ds(rr0, 16)] = cols[d] * m
            return rcarry

        lax.fori_loop(0, NBLK, blk_body, 0)

        for g in range(2):
            pltpu.sync_copy(trans_v.at[g], out_hbm.at[f, g, pl.ds(t0, TW)])
        return carry

    lax.fori_loop(0, FIELD_NUM, field_body, 0)


@jax.jit
def kernel(x, phase, embedding, threshold):
    xt = x.T
    thr = jnp.broadcast_to(threshold, (FIELD_NUM, 16))
    emb4d = embedding.T.reshape(2, 8, RT, 128).transpose(0, 2, 1, 3)
    emb_rm = _sc_relayout(emb4d).reshape(FEATURE_NUM, LATENT_DIM)
    out5 = _sc_embed(xt, thr, emb_rm)
    return out5.transpose(2, 4, 0, 1, 3).reshape(BATCH, FIELD_NUM, LATENT_DIM)


# speedup vs baseline: 1.3317x; 1.3317x over previous
import functools

import jax
import jax.numpy as jnp
from jax import lax
from jax.experimental import pallas as pl
from jax.experimental.pallas import tpu as pltpu
from jax.experimental.pallas import tpu_sc as plsc

FEATURE_NUM = 1040000
LATENT_DIM = 16
FIELD_NUM = 26
BATCH = 16384

NC = 2
NS = 16
NW = NC * NS
BW = BATCH // NW
GATHER_ROWS = 128
KJ = BW // GATHER_ROWS
NBLK = BW // 16
NT = BATCH // 128
TW = NT // NW

_mesh = plsc.VectorSubcoreMesh(core_axis_name="c", subcore_axis_name="s")

RT = FEATURE_NUM // 128


C5 = 5
NCH = RT // C5


@functools.partial(
    pl.kernel,
    out_type=jax.ShapeDtypeStruct((FEATURE_NUM // 8, 128), jnp.float32),
    mesh=_mesh,
    compiler_params=pltpu.CompilerParams(
        needs_layout_passes=False, use_tc_tiling_on_sc=False
    ),
    scratch_types=[
        pltpu.VMEM((2, C5, 8, 128), jnp.float32),
        pltpu.VMEM((2, C5, 8, 128), jnp.float32),
        pltpu.VMEM((16 * C5, 128), jnp.float32),
        pltpu.SemaphoreType.DMA,
        pltpu.SemaphoreType.DMA,
    ],
)
def _sc_relayout(emb4d_hbm, out_hbm, in_a, in_b, out_v, sem_a, sem_b):
    wid = lax.axis_index("s") * NC + lax.axis_index("c")
    c_lo = wid * NCH // NW
    c_hi = (wid + 1) * NCH // NW

    iota = lax.iota(jnp.int32, 16)
    gv = lax.shift_right_logical(iota, 3)
    ddv = lax.bitwise_and(iota, jnp.int32(7))
    bufs = ((in_a, sem_a), (in_b, sem_b))

    def start_in(c, buf, sem):
        @pl.when(c < c_hi)
        def _():
            pltpu.async_copy(emb4d_hbm.at[:, pl.ds(c * C5, C5)], buf, sem)

    for b in range(2):
        start_in(c_lo + b, *bufs[b])

    def pair_body(i2, carry):
        for b in range(2):
            c = c_lo + i2 * 2 + b
            buf, sem = bufs[b]

            @pl.when(c < c_hi)
            def _():
                pltpu.make_async_copy(
                    emb4d_hbm.at[:, pl.ds(0, C5)], buf, sem
                ).wait()
                def j_body(j, jcarry):
                    for tloc in range(C5):
                        tlv = jnp.full((16,), tloc, jnp.int32)
                        for u in range(8):
                            cv = 8 * j + u + jnp.zeros((16,), jnp.int32)
                            out_v[16 * tloc + j, pl.ds(16 * u, 16)] = (
                                plsc.load_gather(buf, [gv, tlv, ddv, cv])
                            )
                    return jcarry

                lax.fori_loop(0, 16, j_body, 0)
                pltpu.sync_copy(
                    out_v, out_hbm.at[pl.ds(c * C5 * 16, C5 * 16)]
                )
                start_in(c + 2, buf, sem)

        return carry

    lax.fori_loop(0, (c_hi - c_lo + 1) // 2, pair_body, 0)


@functools.partial(
    pl.kernel,
    out_type=jax.ShapeDtypeStruct(
        (FIELD_NUM, LATENT_DIM // 8, NT, 8, 128), jnp.float32
    ),
    mesh=_mesh,
    compiler_params=pltpu.CompilerParams(
        needs_layout_passes=False, use_tc_tiling_on_sc=False
    ),
    scratch_types=[
        pltpu.VMEM((BW,), jnp.int32),
        pltpu.VMEM((BW, LATENT_DIM), jnp.float32),
        pltpu.VMEM((2, TW, 8, 128), jnp.float32),
        pltpu.VMEM((FIELD_NUM, 16), jnp.float32),
        pltpu.SemaphoreType.DMA,
    ],
)
def _sc_embed(xt_hbm, thr_hbm, table_hbm, out_hbm, idx_v, rows_v, trans_v,
              thr_v, sem):
    wid = lax.axis_index("s") * NC + lax.axis_index("c")
    b0 = wid * BW
    t0 = wid * TW

    pltpu.sync_copy(thr_hbm, thr_v)

    def field_body(f, carry):
        pltpu.sync_copy(xt_hbm.at[f, pl.ds(b0, BW)], idx_v)

        descs = []
        for j in range(KJ):
            descs.append(
                pltpu.async_copy(
                    table_hbm.at[idx_v.at[pl.ds(j * GATHER_ROWS, GATHER_ROWS)]],
                    rows_v.at[pl.ds(j * GATHER_ROWS, GATHER_ROWS)],
                    sem,
                )
            )
        for d in descs:
            d.wait()

        t_vec = thr_v[f, :]

        def blk_body(blk, rcarry):
            ridx = blk * 16 + lax.iota(jnp.int32, 16)
            sums = jnp.zeros((16,), jnp.float32)
            cols = []
            for d in range(LATENT_DIM):
                cidx = jnp.full((16,), d, jnp.int32)
                col = plsc.load_gather(rows_v, [ridx, cidx])
                cols.append(col)
                sums = sums + jnp.abs(col)
            m = ((sums - t_vec) > 0).astype(jnp.float32)
            tl = blk // 8
            rr0 = (blk % 8) * 16
            for d in range(LATENT_DIM):
                trans_v[d // 8, tl, d % 8, pl.ds(rr0, 16)] = cols[d] * m
            return rcarry

        lax.fori_loop(0, NBLK, blk_body, 0)

        for g in range(2):
            pltpu.sync_copy(trans_v.at[g], out_hbm.at[f, g, pl.ds(t0, TW)])
        return carry

    lax.fori_loop(0, FIELD_NUM, field_body, 0)


@jax.jit
def kernel(x, phase, embedding, threshold):
    xt = x.T
    thr = jnp.broadcast_to(threshold, (FIELD_NUM, 16))
    emb4d = embedding.T.reshape(2, 8, RT, 128).transpose(0, 2, 1, 3)
    emb_rm = _sc_relayout(emb4d).reshape(FEATURE_NUM, LATENT_DIM)
    out5 = _sc_embed(xt, thr, emb_rm)
    return out5.transpose(2, 4, 0, 1, 3).reshape(BATCH, FIELD_NUM, LATENT_DIM)


# speedup vs baseline: 2.2833x; 1.7146x over previous
import functools

import jax
import jax.numpy as jnp
from jax import lax
from jax.experimental import pallas as pl
from jax.experimental.pallas import tpu as pltpu
from jax.experimental.pallas import tpu_sc as plsc

FEATURE_NUM = 1040000
LATENT_DIM = 16
FIELD_NUM = 26
BATCH = 16384

NC = 2
NS = 16
NW = NC * NS
BW = BATCH // NW
NT = BATCH // 128
TW = NT // NW
RT = FEATURE_NUM // 128
GSTRIDE = RT * 1024

_mesh = plsc.VectorSubcoreMesh(core_axis_name="c", subcore_axis_name="s")


@functools.partial(
    pl.kernel,
    out_type=jax.ShapeDtypeStruct(
        (FIELD_NUM, LATENT_DIM // 8, NT, 8, 128), jnp.float32
    ),
    mesh=_mesh,
    compiler_params=pltpu.CompilerParams(
        needs_layout_passes=False, use_tc_tiling_on_sc=False
    ),
    scratch_types=[
        pltpu.VMEM((BW,), jnp.int32),
        pltpu.VMEM((BW,), jnp.int32),
        pltpu.VMEM((TW, LATENT_DIM, 128), jnp.int32),
        pltpu.VMEM((2, TW, 8, 128), jnp.float32),
        pltpu.VMEM((FIELD_NUM, 16), jnp.float32),
        pltpu.SemaphoreType.DMA,
        pltpu.SemaphoreType.DMA,
        pltpu.SemaphoreType.DMA,
        pltpu.SemaphoreType.DMA,
    ],
)
def _sc_embed(xt_hbm, thr_hbm, tabf_hbm, out_hbm, idx_v, base_v, addr_v,
              trans_v, thr_v, sem0, sem1, sem2, sem3):
    wid = lax.axis_index("s") * NC + lax.axis_index("c")
    b0 = wid * BW
    t0 = wid * TW
    sems = (sem0, sem1, sem2, sem3)

    pltpu.sync_copy(thr_hbm, thr_v)

    def field_body(f, carry):
        pltpu.sync_copy(xt_hbm.at[f, pl.ds(b0, BW)], idx_v)

        for q in range(BW // 16):
            v = idx_v[pl.ds(16 * q, 16)]
            base_v[pl.ds(16 * q, 16)] = (
                lax.shift_left(lax.shift_right_logical(v, 7), 10)
                + lax.bitwise_and(v, jnp.int32(127))
            )

        def build_and_fire(tl):
            def d_body(d, dcarry):
                off = (
                    lax.shift_right_logical(d, 3) * GSTRIDE
                    + lax.bitwise_and(d, jnp.int32(7)) * 128
                )
                for q in range(8):
                    addr_v[tl, d, pl.ds(16 * q, 16)] = (
                        base_v[pl.ds(tl * 128 + 16 * q, 16)] + off
                    )
                return dcarry

            lax.fori_loop(0, LATENT_DIM, d_body, 0)
            for g in range(2):
                for dd in range(8):
                    pltpu.async_copy(
                        tabf_hbm.at[addr_v.at[tl, 8 * g + dd]],
                        trans_v.at[g, tl, dd],
                        sems[tl],
                    )

        def drain(tl):
            for _ in range(16):
                pltpu.make_async_copy(
                    tabf_hbm.at[addr_v.at[0, 0]],
                    trans_v.at[0, 0, 0],
                    sems[tl],
                ).wait()

        def compute(tl):
            t_vec = thr_v[f, :]
            zeros = (jnp.zeros((16,), jnp.float32),) * 8

            def sum_body(d, sums):
                return tuple(
                    sums[q]
                    + jnp.abs(
                        trans_v[
                            lax.shift_right_logical(d, 3),
                            tl,
                            lax.bitwise_and(d, jnp.int32(7)),
                            pl.ds(16 * q, 16),
                        ]
                    )
                    for q in range(8)
                )

            sums = lax.fori_loop(0, LATENT_DIM, sum_body, zeros)
            masks = tuple(
                ((sums[q] - t_vec) > 0).astype(jnp.float32) for q in range(8)
            )

            def apply_body(d, dcarry):
                g = lax.shift_right_logical(d, 3)
                dd = lax.bitwise_and(d, jnp.int32(7))
                for q in range(8):
                    trans_v[g, tl, dd, pl.ds(16 * q, 16)] = (
                        trans_v[g, tl, dd, pl.ds(16 * q, 16)] * masks[q]
                    )
                return dcarry

            lax.fori_loop(0, LATENT_DIM, apply_body, 0)

        build_and_fire(0)
        build_and_fire(1)
        drain(0)
        compute(0)
        build_and_fire(2)
        drain(1)
        compute(1)
        build_and_fire(3)
        drain(2)
        compute(2)
        drain(3)
        compute(3)

        for g in range(2):
            pltpu.sync_copy(trans_v.at[g], out_hbm.at[f, g, pl.ds(t0, TW)])
        return carry

    lax.fori_loop(0, FIELD_NUM, field_body, 0)


@jax.jit
def kernel(x, phase, embedding, threshold):
    xt = x.T
    thr = jnp.broadcast_to(threshold, (FIELD_NUM, 16))
    tabf = (
        embedding.T.reshape(2, 8, RT, 128).transpose(0, 2, 1, 3).reshape(-1)
    )
    out5 = _sc_embed(xt, thr, tabf)
    return out5.transpose(2, 4, 0, 1, 3).reshape(BATCH, FIELD_NUM, LATENT_DIM)


# speedup vs baseline: 2.4038x; 1.0528x over previous
import functools

import jax
import jax.numpy as jnp
from jax import lax
from jax.experimental import pallas as pl
from jax.experimental.pallas import tpu as pltpu
from jax.experimental.pallas import tpu_sc as plsc

FEATURE_NUM = 1040000
LATENT_DIM = 16
FIELD_NUM = 26
BATCH = 16384

NC = 2
NS = 16
NW = NC * NS
BW = BATCH // NW
NT = BATCH // 128
TW = NT // NW
RT = FEATURE_NUM // 128
GSTRIDE = RT * 1024

_mesh = plsc.VectorSubcoreMesh(core_axis_name="c", subcore_axis_name="s")


@functools.partial(
    pl.kernel,
    out_type=jax.ShapeDtypeStruct(
        (FIELD_NUM, LATENT_DIM // 8, NT, 8, 128), jnp.float32
    ),
    mesh=_mesh,
    compiler_params=pltpu.CompilerParams(
        needs_layout_passes=False, use_tc_tiling_on_sc=False
    ),
    scratch_types=[
        pltpu.VMEM((2, BW), jnp.int32),
        pltpu.VMEM((BW,), jnp.int32),
        pltpu.VMEM((TW, LATENT_DIM, 128), jnp.int32),
        pltpu.VMEM((2, 2, TW, 8, 128), jnp.float32),
        pltpu.VMEM((FIELD_NUM, 16), jnp.float32),
        pltpu.SemaphoreType.DMA,
        pltpu.SemaphoreType.DMA,
        pltpu.SemaphoreType.DMA,
        pltpu.SemaphoreType.DMA,
        pltpu.SemaphoreType.DMA,
        pltpu.SemaphoreType.DMA,
        pltpu.SemaphoreType.DMA,
        pltpu.SemaphoreType.DMA,
    ],
)
def _sc_embed(xt_hbm, thr_hbm, tabf_hbm, out_hbm, idx2_v, base_v, addr_v,
              trans2_v, thr_v, sem0, sem1, sem2, sem3, semi0, semi1, semo0,
              semo1):
    wid = lax.axis_index("s") * NC + lax.axis_index("c")
    b0 = wid * BW
    t0 = wid * TW
    sems = (sem0, sem1, sem2, sem3)
    semi = (semi0, semi1)
    semo = (semo0, semo1)

    pltpu.sync_copy(thr_hbm, thr_v)
    pltpu.sync_copy(xt_hbm.at[0, pl.ds(b0, BW)], idx2_v.at[0])

    def phase(f2, p):
        f = f2 * 2 + p
        trans_v = trans2_v.at[p]
        idx_v = idx2_v.at[p]

        @pl.when(f2 > 0)
        def _():
            for g in range(2):
                pltpu.make_async_copy(
                    trans2_v.at[p, g], out_hbm.at[0, g, pl.ds(t0, TW)],
                    semo[p],
                ).wait()

        @pl.when(f + 1 < FIELD_NUM)
        def _():
            pltpu.async_copy(
                xt_hbm.at[f + 1, pl.ds(b0, BW)], idx2_v.at[1 - p],
                semi[1 - p],
            )

        @pl.when(f > 0)
        def _():
            pltpu.make_async_copy(
                xt_hbm.at[0, pl.ds(b0, BW)], idx2_v.at[p], semi[p]
            ).wait()

        for q in range(BW // 16):
            v = idx_v[pl.ds(16 * q, 16)]
            base_v[pl.ds(16 * q, 16)] = (
                lax.shift_left(lax.shift_right_logical(v, 7), 10)
                + lax.bitwise_and(v, jnp.int32(127))
            )

        def build_and_fire(tl):
            def d_body(d, dcarry):
                off = (
                    lax.shift_right_logical(d, 3) * GSTRIDE
                    + lax.bitwise_and(d, jnp.int32(7)) * 128
                )
                for q in range(8):
                    addr_v[tl, d, pl.ds(16 * q, 16)] = (
                        base_v[pl.ds(tl * 128 + 16 * q, 16)] + off
                    )
                return dcarry

            lax.fori_loop(0, LATENT_DIM, d_body, 0)
            for g in range(2):
                for dd in range(8):
                    pltpu.async_copy(
                        tabf_hbm.at[addr_v.at[tl, 8 * g + dd]],
                        trans_v.at[g, tl, dd],
                        sems[tl],
                    )

        def drain(tl):
            for _ in range(16):
                pltpu.make_async_copy(
                    tabf_hbm.at[addr_v.at[0, 0]],
                    trans_v.at[0, 0, 0],
                    sems[tl],
                ).wait()

        def compute(tl):
            t_vec = thr_v[f, :]
            zeros = (jnp.zeros((16,), jnp.float32),) * 8

            def sum_body(d, sums):
                return tuple(
                    sums[q]
                    + jnp.abs(
                        trans_v[
                            lax.shift_right_logical(d, 3),
                            tl,
                            lax.bitwise_and(d, jnp.int32(7)),
                            pl.ds(16 * q, 16),
                        ]
                    )
                    for q in range(8)
                )

            sums = lax.fori_loop(0, LATENT_DIM, sum_body, zeros)
            masks = tuple(
                ((sums[q] - t_vec) > 0).astype(jnp.float32) for q in range(8)
            )

            def apply_body(d, dcarry):
                g = lax.shift_right_logical(d, 3)
                dd = lax.bitwise_and(d, jnp.int32(7))
                for q in range(8):
                    trans_v[g, tl, dd, pl.ds(16 * q, 16)] = (
                        trans_v[g, tl, dd, pl.ds(16 * q, 16)] * masks[q]
                    )
                return dcarry

            lax.fori_loop(0, LATENT_DIM, apply_body, 0)

        build_and_fire(0)
        build_and_fire(1)
        drain(0)
        compute(0)
        build_and_fire(2)
        drain(1)
        compute(1)
        build_and_fire(3)
        drain(2)
        compute(2)
        drain(3)
        compute(3)

        for g in range(2):
            pltpu.async_copy(
                trans_v.at[g], out_hbm.at[f, g, pl.ds(t0, TW)], semo[p]
            )

    def pair_body(f2, carry):
        phase(f2, 0)
        phase(f2, 1)
        return carry

    lax.fori_loop(0, FIELD_NUM // 2, pair_body, 0)

    for p in range(2):
        for g in range(2):
            pltpu.make_async_copy(
                trans2_v.at[p, g], out_hbm.at[0, g, pl.ds(t0, TW)], semo[p]
            ).wait()


@jax.jit
def kernel(x, phase, embedding, threshold):
    xt = x.T
    thr = jnp.broadcast_to(threshold, (FIELD_NUM, 16))
    tabf = (
        embedding.T.reshape(2, 8, RT, 128).transpose(0, 2, 1, 3).reshape(-1)
    )
    out5 = _sc_embed(xt, thr, tabf)
    return out5.transpose(2, 4, 0, 1, 3).reshape(BATCH, FIELD_NUM, LATENT_DIM)


# speedup vs baseline: 2.6059x; 1.0841x over previous
import functools

import jax
import jax.numpy as jnp
from jax import lax
from jax.experimental import pallas as pl
from jax.experimental.pallas import tpu as pltpu
from jax.experimental.pallas import tpu_sc as plsc

FEATURE_NUM = 1040000
LATENT_DIM = 16
FIELD_NUM = 26
BATCH = 16384

NC = 2
NS = 16
NW = NC * NS
BW = BATCH // NW
NT = BATCH // 128
TW = NT // NW
RT = FEATURE_NUM // 128
GSTRIDE = RT * 1024

_mesh = plsc.VectorSubcoreMesh(core_axis_name="c", subcore_axis_name="s")


@functools.partial(
    pl.kernel,
    out_type=jax.ShapeDtypeStruct(
        (FIELD_NUM, LATENT_DIM // 8, NT, 8, 128), jnp.float32
    ),
    mesh=_mesh,
    compiler_params=pltpu.CompilerParams(
        needs_layout_passes=False, use_tc_tiling_on_sc=False
    ),
    scratch_types=[
        pltpu.VMEM((2, BW), jnp.int32),
        pltpu.VMEM((BW,), jnp.int32),
        pltpu.VMEM((2, TW, LATENT_DIM, 128), jnp.int32),
        pltpu.VMEM((2, 2, TW, 8, 128), jnp.float32),
        pltpu.VMEM((FIELD_NUM, 16), jnp.float32),
        pltpu.SemaphoreType.DMA,
        pltpu.SemaphoreType.DMA,
        pltpu.SemaphoreType.DMA,
        pltpu.SemaphoreType.DMA,
        pltpu.SemaphoreType.DMA,
        pltpu.SemaphoreType.DMA,
        pltpu.SemaphoreType.DMA,
        pltpu.SemaphoreType.DMA,
    ],
)
def _sc_embed(xt_hbm, thr_hbm, tabf_hbm, out_hbm, idx2_v, base_v, addr_v,
              trans2_v, thr_v, sem0, sem1, sem2, sem3, semi0, semi1, semo0,
              semo1):
    wid = lax.axis_index("s") * NC + lax.axis_index("c")
    b0 = wid * BW
    t0 = wid * TW
    sems = (sem0, sem1, sem2, sem3)
    semi = (semi0, semi1)
    semo = (semo0, semo1)

    pltpu.sync_copy(thr_hbm, thr_v)
    pltpu.sync_copy(xt_hbm.at[0, pl.ds(b0, BW)], idx2_v.at[0])

    def build_and_fire(f, p, tl):
        def d_body(d, dcarry):
            off = (
                lax.shift_right_logical(d, 3) * GSTRIDE
                + lax.bitwise_and(d, jnp.int32(7)) * 128
            )
            for q in range(8):
                addr_v[p, tl, d, pl.ds(16 * q, 16)] = (
                    base_v[pl.ds(tl * 128 + 16 * q, 16)] + off
                )
            return dcarry

        lax.fori_loop(0, LATENT_DIM, d_body, 0)
        for g in range(2):
            for dd in range(8):
                pltpu.async_copy(
                    tabf_hbm.at[addr_v.at[p, tl, 8 * g + dd]],
                    trans2_v.at[p, g, tl, dd],
                    sems[tl],
                )

    def drain(tl):
        for _ in range(16):
            pltpu.make_async_copy(
                tabf_hbm.at[addr_v.at[0, 0, 0]],
                trans2_v.at[0, 0, 0, 0],
                sems[tl],
            ).wait()

    def compute(f, p, tl):
        t_vec = thr_v[f, :]
        zeros = (jnp.zeros((16,), jnp.float32),) * 8

        def sum_body(d, sums):
            return tuple(
                sums[q]
                + jnp.abs(
                    trans2_v[
                        p,
                        lax.shift_right_logical(d, 3),
                        tl,
                        lax.bitwise_and(d, jnp.int32(7)),
                        pl.ds(16 * q, 16),
                    ]
                )
                for q in range(8)
            )

        sums = lax.fori_loop(0, LATENT_DIM, sum_body, zeros)
        masks = tuple(
            ((sums[q] - t_vec) > 0).astype(jnp.float32) for q in range(8)
        )

        def apply_body(d, dcarry):
            g = lax.shift_right_logical(d, 3)
            dd = lax.bitwise_and(d, jnp.int32(7))
            for q in range(8):
                trans2_v[p, g, tl, dd, pl.ds(16 * q, 16)] = (
                    trans2_v[p, g, tl, dd, pl.ds(16 * q, 16)] * masks[q]
                )
            return dcarry

        lax.fori_loop(0, LATENT_DIM, apply_body, 0)

    def prelude(f, p, first):
        if not first:
            @pl.when(f >= 2)
            def _():
                for g in range(2):
                    pltpu.make_async_copy(
                        trans2_v.at[p, g], out_hbm.at[0, g, pl.ds(t0, TW)],
                        semo[p],
                    ).wait()

        def _prefetch():
            pltpu.async_copy(
                xt_hbm.at[f + 1, pl.ds(b0, BW)], idx2_v.at[1 - p],
                semi[1 - p],
            )

        if first:
            _prefetch()
        else:
            pl.when(f + 1 < FIELD_NUM)(_prefetch)

        if not first:
            pltpu.make_async_copy(
                xt_hbm.at[0, pl.ds(b0, BW)], idx2_v.at[p], semi[p]
            ).wait()

        for q in range(BW // 16):
            v = idx2_v[p, pl.ds(16 * q, 16)]
            base_v[pl.ds(16 * q, 16)] = (
                lax.shift_left(lax.shift_right_logical(v, 7), 10)
                + lax.bitwise_and(v, jnp.int32(127))
            )

        build_and_fire(f, p, 0)
        build_and_fire(f, p, 1)

    def body(f2, p):
        f = f2 * 2 + p
        build_and_fire(f, p, 2)
        drain(0)
        compute(f, p, 0)
        build_and_fire(f, p, 3)
        drain(1)
        compute(f, p, 1)
        if p == 0:
            prelude(f + 1, 1, False)
        else:
            @pl.when(f2 + 1 < FIELD_NUM // 2)
            def _():
                prelude(f + 1, 0, False)
        drain(2)
        compute(f, p, 2)
        drain(3)
        compute(f, p, 3)
        for g in range(2):
            pltpu.async_copy(
                trans2_v.at[p, g], out_hbm.at[f, g, pl.ds(t0, TW)], semo[p]
            )

    prelude(0, 0, True)

    def pair_body(f2, carry):
        body(f2, 0)
        body(f2, 1)
        return carry

    lax.fori_loop(0, FIELD_NUM // 2, pair_body, 0)

    for p in range(2):
        for g in range(2):
            pltpu.make_async_copy(
                trans2_v.at[p, g], out_hbm.at[0, g, pl.ds(t0, TW)], semo[p]
            ).wait()


@jax.jit
def kernel(x, phase, embedding, threshold):
    xt = x.T
    thr = jnp.broadcast_to(threshold, (FIELD_NUM, 16))
    tabf = (
        embedding.T.reshape(2, 8, RT, 128).transpose(0, 2, 1, 3).reshape(-1)
    )
    out5 = _sc_embed(xt, thr, tabf)
    return out5.transpose(2, 4, 0, 1, 3).reshape(BATCH, FIELD_NUM, LATENT_DIM)


# speedup vs baseline: 2.6165x; 1.0041x over previous
import functools

import jax
import jax.numpy as jnp
from jax import lax
from jax.experimental import pallas as pl
from jax.experimental.pallas import tpu as pltpu
from jax.experimental.pallas import tpu_sc as plsc

FEATURE_NUM = 1040000
LATENT_DIM = 16
FIELD_NUM = 26
BATCH = 16384

NC = 2
NS = 16
NW = NC * NS
BW = BATCH // NW
NT = BATCH // 128
TW = NT // NW
RT = FEATURE_NUM // 128
GSTRIDE = RT * 1024

_mesh = plsc.VectorSubcoreMesh(core_axis_name="c", subcore_axis_name="s")


@functools.partial(
    pl.kernel,
    out_type=jax.ShapeDtypeStruct(
        (FIELD_NUM, LATENT_DIM // 8, NT, 8, 128), jnp.float32
    ),
    mesh=_mesh,
    compiler_params=pltpu.CompilerParams(
        needs_layout_passes=False, use_tc_tiling_on_sc=False
    ),
    scratch_types=[
        pltpu.VMEM((2, BW), jnp.int32),
        pltpu.VMEM((BW,), jnp.int32),
        pltpu.VMEM((2, TW, LATENT_DIM, 128), jnp.int32),
        pltpu.VMEM((2, 2, TW, 8, 128), jnp.float32),
        pltpu.VMEM((FIELD_NUM, 16), jnp.float32),
        pltpu.VMEM((16 * 128,), jnp.float32),
        pltpu.SemaphoreType.DMA,
        pltpu.SemaphoreType.DMA,
        pltpu.SemaphoreType.DMA,
        pltpu.SemaphoreType.DMA,
        pltpu.SemaphoreType.DMA,
        pltpu.SemaphoreType.DMA,
        pltpu.SemaphoreType.DMA,
        pltpu.SemaphoreType.DMA,
    ],
)
def _sc_embed(xt_hbm, thr_hbm, tabf_hbm, out_hbm, idx2_v, base_v, addr_v,
              trans2_v, thr_v, drain_v, sem0, sem1, sem2, sem3, semi0, semi1,
              semo0, semo1):
    wid = lax.axis_index("s") * NC + lax.axis_index("c")
    b0 = wid * BW
    t0 = wid * TW
    sems = (sem0, sem1, sem2, sem3)
    semi = (semi0, semi1)
    semo = (semo0, semo1)

    pltpu.sync_copy(thr_hbm, thr_v)
    pltpu.sync_copy(xt_hbm.at[0, pl.ds(b0, BW)], idx2_v.at[0])

    def build_and_fire(f, p, tl):
        def d_body(d, dcarry):
            off = (
                lax.shift_right_logical(d, 3) * GSTRIDE
                + lax.bitwise_and(d, jnp.int32(7)) * 128
            )
            for q in range(8):
                addr_v[p, tl, d, pl.ds(16 * q, 16)] = (
                    base_v[pl.ds(tl * 128 + 16 * q, 16)] + off
                )
            return dcarry

        lax.fori_loop(0, LATENT_DIM, d_body, 0)
        for g in range(2):
            for dd in range(8):
                pltpu.async_copy(
                    tabf_hbm.at[addr_v.at[p, tl, 8 * g + dd]],
                    trans2_v.at[p, g, tl, dd],
                    sems[tl],
                )

    def drain(tl):
        pltpu.make_async_copy(
            tabf_hbm.at[pl.ds(0, 16 * 128)], drain_v, sems[tl]
        ).wait()

    def compute(f, p, tl):
        t_vec = thr_v[f, :]
        zeros = (jnp.zeros((16,), jnp.float32),) * 8

        def sum_body(d, sums):
            return tuple(
                sums[q]
                + jnp.abs(
                    trans2_v[
                        p,
                        lax.shift_right_logical(d, 3),
                        tl,
                        lax.bitwise_and(d, jnp.int32(7)),
                        pl.ds(16 * q, 16),
                    ]
                )
                for q in range(8)
            )

        sums = lax.fori_loop(0, LATENT_DIM, sum_body, zeros)
        masks = tuple(
            ((sums[q] - t_vec) > 0).astype(jnp.float32) for q in range(8)
        )

        def apply_body(d, dcarry):
            g = lax.shift_right_logical(d, 3)
            dd = lax.bitwise_and(d, jnp.int32(7))
            for q in range(8):
                trans2_v[p, g, tl, dd, pl.ds(16 * q, 16)] = (
                    trans2_v[p, g, tl, dd, pl.ds(16 * q, 16)] * masks[q]
                )
            return dcarry

        lax.fori_loop(0, LATENT_DIM, apply_body, 0)

    def prelude(f, p, first):
        if not first:
            @pl.when(f >= 2)
            def _():
                for g in range(2):
                    pltpu.make_async_copy(
                        trans2_v.at[p, g], out_hbm.at[0, g, pl.ds(t0, TW)],
                        semo[p],
                    ).wait()

        def _prefetch():
            pltpu.async_copy(
                xt_hbm.at[f + 1, pl.ds(b0, BW)], idx2_v.at[1 - p],
                semi[1 - p],
            )

        if first:
            _prefetch()
        else:
            pl.when(f + 1 < FIELD_NUM)(_prefetch)

        if not first:
            pltpu.make_async_copy(
                xt_hbm.at[0, pl.ds(b0, BW)], idx2_v.at[p], semi[p]
            ).wait()

        for q in range(BW // 16):
            v = idx2_v[p, pl.ds(16 * q, 16)]
            base_v[pl.ds(16 * q, 16)] = (
                lax.shift_left(lax.shift_right_logical(v, 7), 10)
                + lax.bitwise_and(v, jnp.int32(127))
            )

        build_and_fire(f, p, 0)
        build_and_fire(f, p, 1)

    def body(f2, p):
        f = f2 * 2 + p
        build_and_fire(f, p, 2)
        drain(0)
        compute(f, p, 0)
        build_and_fire(f, p, 3)
        drain(1)
        compute(f, p, 1)
        if p == 0:
            prelude(f + 1, 1, False)
        else:
            @pl.when(f2 + 1 < FIELD_NUM // 2)
            def _():
                prelude(f + 1, 0, False)
        drain(2)
        compute(f, p, 2)
        drain(3)
        compute(f, p, 3)
        for g in range(2):
            pltpu.async_copy(
                trans2_v.at[p, g], out_hbm.at[f, g, pl.ds(t0, TW)], semo[p]
            )

    prelude(0, 0, True)

    def pair_body(f2, carry):
        body(f2, 0)
        body(f2, 1)
        return carry

    lax.fori_loop(0, FIELD_NUM // 2, pair_body, 0)

    for p in range(2):
        for g in range(2):
            pltpu.make_async_copy(
                trans2_v.at[p, g], out_hbm.at[0, g, pl.ds(t0, TW)], semo[p]
            ).wait()


@jax.jit
def kernel(x, phase, embedding, threshold):
    xt = x.T
    thr = jnp.broadcast_to(threshold, (FIELD_NUM, 16))
    tabf = (
        embedding.T.reshape(2, 8, RT, 128).transpose(0, 2, 1, 3).reshape(-1)
    )
    out5 = _sc_embed(xt, thr, tabf)
    return out5.transpose(2, 4, 0, 1, 3).reshape(BATCH, FIELD_NUM, LATENT_DIM)
